# baseline (speedup 1.0000x reference)
import functools

import jax
import jax.numpy as jnp
from jax.experimental import pallas as pl

B, D, N, K, NC = 1024, 128, 100000, 64, 8192

NPAD = 100352

_BB = 512
_BN = 2048


def _matmul_body(ctx_ref, w_ref, b_ref, out_ref):
    out_ref[...] = (
        jnp.dot(ctx_ref[...], w_ref[...], preferred_element_type=jnp.float32)
        + b_ref[...]
    )


def _logits_matmul(context, w1p, b1p):
    grid = (B // _BB, NPAD // _BN)
    return pl.pallas_call(
        _matmul_body,
        grid=grid,
        in_specs=[
            pl.BlockSpec((_BB, D), lambda i, j: (i, 0)),
            pl.BlockSpec((D, _BN), lambda i, j: (0, j)),
            pl.BlockSpec((1, _BN), lambda i, j: (0, j)),
        ],
        out_specs=pl.BlockSpec((_BB, _BN), lambda i, j: (i, j)),
        out_shape=jax.ShapeDtypeStruct((B, NPAD), jnp.float32),
    )(context, w1p, b1p)


@jax.jit
def kernel(context, candidates, W1, b1, E2):
    U = jax.random.uniform(jax.random.key(42), (B, NC),
                           minval=1e-6, maxval=1.0, dtype=jnp.float32)
    gumbel = -jnp.log(-jnp.log(U))

    w1p = jnp.pad(W1, ((0, 0), (0, NPAD - N)))
    b1p = jnp.pad(b1, (0, NPAD - N)).reshape(1, NPAD)

    logits = _logits_matmul(context, w1p, b1p)

    cand_logits = jnp.take_along_axis(logits, candidates, axis=1)
    noisy = cand_logits + gumbel
    _, topk_pos = jax.lax.top_k(noisy, K)
    topk_indices = jnp.take_along_axis(candidates, topk_pos, axis=1)

    item_emb = jnp.take(E2, topk_indices, axis=0)
    scores = jnp.einsum('bd,bkd->bk', context, item_emb)
    probs = jax.nn.softmax(scores, axis=1)
    return probs


# speedup vs baseline: 3.0588x; 3.0588x over previous
import jax
import jax.numpy as jnp
from jax import lax
from jax.experimental import pallas as pl
from jax.experimental.pallas import tpu as pltpu
from jax.experimental.pallas import tpu_sc as plsc

B, D, N, K, NC = 1024, 128, 100000, 64, 8192

NPAD = 100352
NROW = NPAD // 128

_BB = 512
_BN = 2048

NW = 32
RPW = B // NW
NBK = 128
CAP = 128
NEG = -3.0e38


def _matmul_body(ctx_ref, w_ref, b_ref, out_ref):
    out_ref[...] = (
        jnp.dot(ctx_ref[...], w_ref[...], preferred_element_type=jnp.float32)
        + b_ref[...]
    )


def _logits_matmul(context, w1p, b1p):
    grid = (B // _BB, NPAD // _BN)
    return pl.pallas_call(
        _matmul_body,
        grid=grid,
        in_specs=[
            pl.BlockSpec((_BB, D), lambda i, j: (i, 0)),
            pl.BlockSpec((D, _BN), lambda i, j: (0, j)),
            pl.BlockSpec((1, _BN), lambda i, j: (0, j)),
        ],
        out_specs=pl.BlockSpec((_BB, _BN), lambda i, j: (i, j)),
        out_shape=jax.ShapeDtypeStruct((B, NPAD), jnp.float32),
    )(context, w1p, b1p)


def _cmpx(ka, ia, kb, ib, a_takes_max):
    m = ka >= kb
    hi_k = jnp.where(m, ka, kb)
    hi_i = jnp.where(m, ia, ib)
    lo_k = jnp.where(m, kb, ka)
    lo_i = jnp.where(m, ib, ia)
    if a_takes_max:
        return hi_k, hi_i, lo_k, lo_i
    return lo_k, lo_i, hi_k, hi_i


def _bitonic128_desc(keys, ids):
    keys = list(keys)
    ids = list(ids)
    for a in range(8):
        keys[a], ids[a] = plsc.sort_key_val(
            keys[a], ids[a], descending=(a % 2 == 0))
    for kk in (32, 64, 128):
        j = kk // 2
        while j >= 16:
            jv = j // 16
            for a in range(8):
                if a & jv:
                    continue
                p = a ^ jv
                a_max = ((a * 16) & kk) == 0
                keys[a], ids[a], keys[p], ids[p] = _cmpx(
                    keys[a], ids[a], keys[p], ids[p], a_max)
            j //= 2
        for a in range(8):
            keys[a], ids[a] = plsc.sort_key_val(
                keys[a], ids[a], descending=((a * 16) & kk) == 0)
    return keys, ids


def _make_sc_ranker():
    info = plsc.get_sparse_core_info()
    n_cores = info.num_cores
    mesh = plsc.VectorSubcoreMesh(core_axis_name="c", subcore_axis_name="s")

    def body(logits_r, candf_r, gum_r, ctx_r, e2_r, out_r,
             rowv, candv, noisyv, embv, ctxv, hist, bufk, bufi,
             idsv, probsv, sem0, sem2):
        lane = lax.iota(jnp.int32, 16)
        ones_i = jnp.ones((16,), jnp.int32)
        wid = lax.axis_index("s") * n_cores + lax.axis_index("c")
        base = wid * RPW

        def compute(i, _):
            b = base + i
            pltpu.sync_copy(logits_r.at[b], rowv)
            pltpu.sync_copy(candf_r.at[b], candv)
            pltpu.sync_copy(gum_r.at[b], noisyv)
            pltpu.sync_copy(ctx_r.at[b], ctxv)

            def p1(j, carry):
                mx, mn = carry
                for u in range(4):
                    sl = pl.ds(j * 64 + u * 16, 16)
                    c = candv[sl]
                    g = plsc.load_gather(rowv, [c >> 7, c & 127])
                    v = g + noisyv[sl]
                    noisyv[sl] = v
                    mx = jnp.maximum(mx, v)
                    mn = jnp.minimum(mn, v)
                return mx, mn

            mx, mn = lax.fori_loop(
                0, NC // 64, p1,
                (jnp.full((16,), NEG, jnp.float32),
                 jnp.full((16,), -NEG, jnp.float32)))
            gmax = plsc.cummax(mx)[15]
            gmin = -plsc.cummax(-mn)[15]
            rng_v = jnp.broadcast_to(jnp.maximum(gmax - gmin, 1e-30), (16,))
            scale_v = jnp.full((16,), float(NBK), jnp.float32) / rng_v

            def bucket(v):
                t = (gmax - v) * scale_v
                return jnp.clip(t.astype(jnp.int32), 0, NBK - 1)

            def pz(j, _):
                hist[pl.ds(j * 16, 16)] = jnp.zeros((16,), jnp.int32)
                return 0

            lax.fori_loop(0, NBK, pz, 0, unroll=4)

            def p2(j, _):
                for u in range(4):
                    sl = pl.ds(j * 64 + u * 16, 16)
                    bidx = bucket(noisyv[sl])
                    plsc.addupdate_scatter(hist, [lane * NBK + bidx], ones_i)
                return 0

            lax.fori_loop(0, NC // 64, p2, 0)

            T_v = jnp.full((16,), NBK - 1, jnp.int32)
            found_v = jnp.zeros((16,), jnp.bool_)
            prev = jnp.int32(0)
            for g16 in range(NBK // 16):
                tot = hist[pl.ds(g16 * 16, 16)]
                for L in range(1, 16):
                    tot = tot + hist[pl.ds(L * NBK + g16 * 16, 16)]
                c = plsc.cumsum(tot) + prev
                ge = c >= K
                npos = plsc.all_reduce_population_count(ge)
                ffs = plsc.all_reduce_ffs(ge)
                has_v = npos > 0
                T_v = jnp.where(found_v, T_v,
                                jnp.where(has_v, g16 * 16 + ffs, T_v))
                found_v = jnp.logical_or(found_v, has_v)
                prev = c[15]

            def p4(j, cnt):
                for u in range(4):
                    sl = pl.ds(j * 64 + u * 16, 16)
                    v = noisyv[sl]
                    keep = bucket(v) <= T_v
                    pop = plsc.all_reduce_population_count(keep)
                    off = jnp.minimum(cnt, CAP)
                    plsc.store_compressed(bufk.at[pl.ds(off, 16)], v,
                                          mask=keep)
                    plsc.store_compressed(bufi.at[pl.ds(off, 16)],
                                          candv[sl], mask=keep)
                    cnt = cnt + pop[0]
                return cnt

            cnt = lax.fori_loop(0, NC // 64, p4, jnp.int32(0))

            for w in range(9):
                sl = pl.ds(w * 16, 16)
                kv = bufk[sl]
                bufk[sl] = jnp.where(w * 16 + lane < cnt, kv, NEG)

            keys = [bufk[pl.ds(a * 16, 16)] for a in range(8)]
            ids = [bufi[pl.ds(a * 16, 16)] for a in range(8)]
            keys, ids = _bitonic128_desc(keys, ids)

            for g in range(4):
                idsv[pl.ds(g * 16, 16)] = ids[g]

            pltpu.async_copy(e2_r.at[idsv], embv, sem2).wait()

            cs = [ctxv[pl.ds(dv * 16, 16)] for dv in range(8)]
            svs = [jnp.zeros((16,), jnp.float32) for _ in range(4)]
            for k in range(K):
                acc = embv[k, pl.ds(0, 16)] * cs[0]
                for dv in range(1, 8):
                    acc = acc + embv[k, pl.ds(dv * 16, 16)] * cs[dv]
                s = plsc.cumsum(acc)[15]
                svs[k // 16] = svs[k // 16] + jnp.where(
                    lane == (k % 16), s, 0.0)

            m = plsc.cummax(jnp.maximum(jnp.maximum(svs[0], svs[1]),
                                        jnp.maximum(svs[2], svs[3])))[15]
            es = [jnp.exp(sv - m) for sv in svs]
            tot_v = jnp.broadcast_to(
                plsc.cumsum(es[0] + es[1] + es[2] + es[3])[15], (16,))
            for g in range(4):
                probsv[pl.ds(g * 16, 16)] = es[g] / tot_v
            pltpu.sync_copy(probsv, out_r.at[b])
            return 0

        lax.fori_loop(0, RPW, compute, 0)

    return pl.kernel(
        body,
        out_type=jax.ShapeDtypeStruct((B, K), jnp.float32),
        mesh=mesh,
        compiler_params=pltpu.CompilerParams(needs_layout_passes=False),
        scratch_types=[
            pltpu.VMEM((NROW, 128), jnp.float32),
            pltpu.VMEM((NC,), jnp.int32),
            pltpu.VMEM((NC,), jnp.float32),
            pltpu.VMEM((K, D), jnp.float32),
            pltpu.VMEM((D,), jnp.float32),
            pltpu.VMEM((NBK * 16,), jnp.int32),
            pltpu.VMEM((CAP + 16,), jnp.float32),
            pltpu.VMEM((CAP + 16,), jnp.int32),
            pltpu.VMEM((K,), jnp.int32),
            pltpu.VMEM((K,), jnp.float32),
            pltpu.SemaphoreType.DMA,
            pltpu.SemaphoreType.DMA,
        ],
    )


@jax.jit
def kernel(context, candidates, W1, b1, E2):
    U = jax.random.uniform(jax.random.key(42), (B, NC),
                           minval=1e-6, maxval=1.0, dtype=jnp.float32)
    gumbel = -jnp.log(-jnp.log(U))

    w1p = jnp.pad(W1, ((0, 0), (0, NPAD - N)))
    b1p = jnp.pad(b1, (0, NPAD - N)).reshape(1, NPAD)

    logits = _logits_matmul(context, w1p, b1p)

    ranker = _make_sc_ranker()
    return ranker(logits.reshape(B, NROW, 128), candidates, gumbel,
                  context, E2)
